# baseline (speedup 1.0000x reference)
import functools

import jax
import jax.numpy as jnp
import numpy as np
from jax.experimental import pallas as pl
from jax.experimental.pallas import tpu as pltpu

E = 100
HX = 128
HD = 64
NHX = 6
NHD = 4
B = 4096
DIMX = 3 * NHX * 2
DIMD = 3 * NHD * 2

T = 128
NT = B // T + E
NS = NT * T

_OFF = [0, 40, 168, 296, 424, 552, 720, 848]
_DIN = [DIMX, HX, HX, HX, HX, HX + DIMX, HX, HX]
_OFF_INT = 976
_R1 = 1104
_R2 = 152 + 64 + 128


def _harm_tile(v, n):
    f = jnp.exp2(jax.lax.broadcasted_iota(jnp.int32, (1, n), 1).astype(jnp.float32))
    cols = [v[:, i:i + 1] * f for i in range(3)]
    e = jnp.concatenate(cols, axis=-1)
    return jnp.concatenate([jnp.sin(e), jnp.cos(e)], axis=-1)


def _moe_body(sp_ref, xds_ref, w1_ref, w2_ref, wb_ref, out_ref):
    t = pl.program_id(0)
    a = sp_ref[NT + t]
    xdt = xds_ref[pl.ds(a, T), :]
    ex = _harm_tile(xdt[:, 0:3], NHX)
    ed = _harm_tile(xdt[:, 3:6], NHD)
    w = w1_ref[0]
    b = wb_ref[0]
    y = ex
    for li in range(8):
        if li == 5:
            y = jnp.concatenate([y, ex], axis=-1)
        wli = w[_OFF[li]:_OFF[li] + _DIN[li]]
        y = jnp.maximum(
            jnp.dot(y, wli, preferred_element_type=jnp.float32) + b[li:li + 1], 0.0)
    inter = jnp.dot(y, w[_OFF_INT:_OFF_INT + HX],
                    preferred_element_type=jnp.float32) + b[8:9]
    w2 = w2_ref[0]
    density = jnp.dot(y, w2[216:344, 0:1],
                      preferred_element_type=jnp.float32) + b[9:10, 0:1]
    ci = jnp.concatenate([inter, ed], axis=-1)
    c = jnp.maximum(
        jnp.dot(ci, w2[0:152], preferred_element_type=jnp.float32) + b[10:11, 0:HD],
        0.0)
    color = jax.nn.sigmoid(
        jnp.dot(c, w2[152:216, 0:3], preferred_element_type=jnp.float32)
        + b[11:12, 0:3])
    out_ref[:] = jnp.concatenate([density, color], axis=-1)


def _pad_rows(w, rows):
    return jnp.pad(w, ((0, 0), (0, rows - w.shape[1]), (0, 0)))


def _pad_lanes(bvec, lanes=HX):
    return jnp.pad(bvec, ((0, 0), (0, lanes - bvec.shape[1])))


@jax.jit
def kernel(x, d, index, wx0, bx0, wx1, bx1, wx2, bx2, wx3, bx3, wx4, bx4,
           wx5, bx5, wx6, bx6, wx7, bx7, wint, bint, wden, bden, wc1, bc1,
           wc2, bc2):
    w1 = jnp.concatenate(
        [_pad_rows(wx0, 40), wx1, wx2, wx3, wx4, _pad_rows(wx5, 168),
         wx6, wx7, wint], axis=1)
    w2 = jnp.concatenate(
        [wc1, jnp.pad(wc2, ((0, 0), (0, 0), (0, HD - 3))),
         jnp.pad(wden, ((0, 0), (0, 0), (0, HD - 1)))], axis=1)
    wb = jnp.stack(
        [bx0, bx1, bx2, bx3, bx4, bx5, bx6, bx7, bint,
         _pad_lanes(bden), _pad_lanes(bc1), _pad_lanes(bc2)], axis=1)

    idx = index.astype(jnp.int32)
    order = jnp.argsort(idx).astype(jnp.int32)
    counts = jnp.bincount(idx, length=E).astype(jnp.int32)
    starts = jnp.concatenate(
        [jnp.zeros((1,), jnp.int32), jnp.cumsum(counts)[:-1].astype(jnp.int32)])
    pad_counts = ((counts + T - 1) // T) * T
    pcsum = jnp.cumsum(pad_counts).astype(jnp.int32)
    pad_starts = pcsum - pad_counts

    tile_e = jnp.searchsorted(
        pcsum, jnp.arange(NT, dtype=jnp.int32) * T, side='right').astype(jnp.int32)
    tile_e = jnp.minimum(tile_e, E - 1)
    srow = jnp.clip(
        starts[tile_e] + jnp.arange(NT, dtype=jnp.int32) * T - pad_starts[tile_e],
        0, B)
    sp = jnp.concatenate([tile_e, srow])

    xd = jnp.concatenate([x, d], axis=1)
    xds = jnp.concatenate([xd[order], jnp.zeros((T, 6), jnp.float32)], axis=0)

    grid_spec = pltpu.PrefetchScalarGridSpec(
        num_scalar_prefetch=1,
        grid=(NT,),
        in_specs=[
            pl.BlockSpec((B + T, 6), lambda t, sp: (0, 0)),
            pl.BlockSpec((1, _R1, HX), lambda t, sp: (sp[t], 0, 0)),
            pl.BlockSpec((1, _R2, HD), lambda t, sp: (sp[t], 0, 0)),
            pl.BlockSpec((1, 12, HX), lambda t, sp: (sp[t], 0, 0)),
        ],
        out_specs=pl.BlockSpec((T, 4), lambda t, sp: (t, 0)),
    )
    outp = pl.pallas_call(
        _moe_body,
        grid_spec=grid_spec,
        out_shape=jax.ShapeDtypeStruct((NS, 4), jnp.float32),
    )(sp, xds, w1, w2, wb)

    sorted_e = idx[order]
    slot = jnp.arange(B, dtype=jnp.int32) + (pad_starts - starts)[sorted_e]
    slot_of_row = jnp.zeros((B,), jnp.int32).at[order].set(slot)
    return outp[slot_of_row]


# speedup vs baseline: 1.0946x; 1.0946x over previous
import functools

import jax
import jax.numpy as jnp
import numpy as np
from jax.experimental import pallas as pl
from jax.experimental.pallas import tpu as pltpu

E = 100
HX = 128
HD = 64
NHX = 6
NHD = 4
B = 4096
DIMX = 3 * NHX * 2
DIMD = 3 * NHD * 2

T = 128
NT = B // T + E
NS = NT * T

_OFF = [0, 40, 168, 296, 424, 552, 720, 848]
_DIN = [DIMX, HX, HX, HX, HX, HX + DIMX, HX, HX]
_OFF_INT = 976
_R1 = 1104
_R2 = 152 + 64 + 128


def _mlp_chain(emb, w, w2, b):
    ex = emb[:, 0:DIMX]
    ed = emb[:, DIMX:DIMX + DIMD]
    y = ex
    for li in range(8):
        if li == 5:
            y = jnp.concatenate([y, ex], axis=-1)
        wli = w[_OFF[li]:_OFF[li] + _DIN[li]]
        y = jnp.maximum(
            jnp.dot(y, wli, preferred_element_type=jnp.float32) + b[li:li + 1], 0.0)
    inter = jnp.dot(y, w[_OFF_INT:_OFF_INT + HX],
                    preferred_element_type=jnp.float32) + b[8:9]
    density = jnp.dot(y, w2[216:344, 0:1],
                      preferred_element_type=jnp.float32) + b[9:10, 0:1]
    ci = jnp.concatenate([inter, ed], axis=-1)
    c = jnp.maximum(
        jnp.dot(ci, w2[0:152], preferred_element_type=jnp.float32) + b[10:11, 0:HD],
        0.0)
    color = jax.nn.sigmoid(
        jnp.dot(c, w2[152:216, 0:3], preferred_element_type=jnp.float32)
        + b[11:12, 0:3])
    return jnp.concatenate([density, color], axis=-1)


def _moe_body(sp_ref, exds_ref, w1_ref, w2_ref, wb_ref, out_ref):
    t = pl.program_id(0)
    a = sp_ref[NT + t]
    emb = exds_ref[pl.ds(a, T), :]
    w = w1_ref[0]
    w2 = w2_ref[0]
    b = wb_ref[0]
    out_ref[:] = _mlp_chain(emb, w, w2, b)


def _pad_rows(w, rows):
    return jnp.pad(w, ((0, 0), (0, rows - w.shape[1]), (0, 0)))


def _pad_lanes(bvec, lanes=HX):
    return jnp.pad(bvec, ((0, 0), (0, lanes - bvec.shape[1])))


@jax.jit
def kernel(x, d, index, wx0, bx0, wx1, bx1, wx2, bx2, wx3, bx3, wx4, bx4,
           wx5, bx5, wx6, bx6, wx7, bx7, wint, bint, wden, bden, wc1, bc1,
           wc2, bc2):
    w1 = jnp.concatenate(
        [_pad_rows(wx0, 40), wx1, wx2, wx3, wx4, _pad_rows(wx5, 168),
         wx6, wx7, wint], axis=1)
    w2 = jnp.concatenate(
        [wc1, jnp.pad(wc2, ((0, 0), (0, 0), (0, HD - 3))),
         jnp.pad(wden, ((0, 0), (0, 0), (0, HD - 1)))], axis=1)
    wb = jnp.stack(
        [bx0, bx1, bx2, bx3, bx4, bx5, bx6, bx7, bint,
         _pad_lanes(bden), _pad_lanes(bc1), _pad_lanes(bc2)], axis=1)

    idx = index.astype(jnp.int32)
    order = jnp.argsort(idx).astype(jnp.int32)
    counts = jnp.bincount(idx, length=E).astype(jnp.int32)
    starts = jnp.concatenate(
        [jnp.zeros((1,), jnp.int32), jnp.cumsum(counts)[:-1].astype(jnp.int32)])
    pad_counts = ((counts + T - 1) // T) * T
    pcsum = jnp.cumsum(pad_counts).astype(jnp.int32)
    pad_starts = pcsum - pad_counts

    tile_e = jnp.searchsorted(
        pcsum, jnp.arange(NT, dtype=jnp.int32) * T, side='right').astype(jnp.int32)
    tile_e = jnp.minimum(tile_e, E - 1)
    srow = jnp.clip(
        starts[tile_e] + jnp.arange(NT, dtype=jnp.int32) * T - pad_starts[tile_e],
        0, B)
    sp = jnp.concatenate([tile_e, srow])

    xd = jnp.concatenate([x, d], axis=1)
    xds = jnp.concatenate([xd[order], jnp.zeros((T, 6), jnp.float32)], axis=0)
    fx = 2.0 ** jnp.arange(NHX, dtype=jnp.float32)
    fd = 2.0 ** jnp.arange(NHD, dtype=jnp.float32)
    px = (xds[:, 0:3, None] * fx).reshape(B + T, 3 * NHX)
    pd = (xds[:, 3:6, None] * fd).reshape(B + T, 3 * NHD)
    exds = jnp.concatenate(
        [jnp.sin(px), jnp.cos(px), jnp.sin(pd), jnp.cos(pd)], axis=1)

    grid_spec = pltpu.PrefetchScalarGridSpec(
        num_scalar_prefetch=1,
        grid=(NT,),
        in_specs=[
            pl.BlockSpec((B + T, 60), lambda t, sp: (0, 0)),
            pl.BlockSpec((1, _R1, HX), lambda t, sp: (sp[t], 0, 0)),
            pl.BlockSpec((1, _R2, HD), lambda t, sp: (sp[t], 0, 0)),
            pl.BlockSpec((1, 12, HX), lambda t, sp: (sp[t], 0, 0)),
        ],
        out_specs=pl.BlockSpec((T, 4), lambda t, sp: (t, 0)),
    )
    outp = pl.pallas_call(
        _moe_body,
        grid_spec=grid_spec,
        out_shape=jax.ShapeDtypeStruct((NS, 4), jnp.float32),
    )(sp, exds, w1, w2, wb)

    sorted_e = idx[order]
    slot = jnp.arange(B, dtype=jnp.int32) + (pad_starts - starts)[sorted_e]
    slot_of_row = jnp.zeros((B,), jnp.int32).at[order].set(slot)
    return outp[slot_of_row]


# speedup vs baseline: 1.1958x; 1.0925x over previous
import functools

import jax
import jax.numpy as jnp
import numpy as np
from jax.experimental import pallas as pl
from jax.experimental.pallas import tpu as pltpu

E = 100
HX = 128
HD = 64
NHX = 6
NHD = 4
B = 4096
DIMX = 3 * NHX * 2
DIMD = 3 * NHD * 2

T = 128
NT = B // T + E
NS = NT * T

_OFF = [0, 40, 168, 296, 424, 552, 720, 848]
_DIN = [DIMX, HX, HX, HX, HX, HX + DIMX, HX, HX]
_OFF_INT = 976
_R1 = 1104
_R2 = 152 + 64 + 128


def _mlp_chain(emb, w, w2, b):
    ex = emb[:, 0:DIMX]
    ed = emb[:, DIMX:DIMX + DIMD]
    y = ex
    for li in range(8):
        if li == 5:
            y = jnp.concatenate([y, ex], axis=-1)
        wli = w[_OFF[li]:_OFF[li] + _DIN[li]]
        y = jnp.maximum(
            jnp.dot(y, wli, preferred_element_type=jnp.float32) + b[li:li + 1], 0.0)
    inter = jnp.dot(y, w[_OFF_INT:_OFF_INT + HX],
                    preferred_element_type=jnp.float32) + b[8:9]
    density = jnp.dot(y, w2[216:344, 0:1],
                      preferred_element_type=jnp.float32) + b[9:10, 0:1]
    ci = jnp.concatenate([inter, ed], axis=-1)
    c = jnp.maximum(
        jnp.dot(ci, w2[0:152], preferred_element_type=jnp.float32) + b[10:11, 0:HD],
        0.0)
    color = jax.nn.sigmoid(
        jnp.dot(c, w2[152:216, 0:3], preferred_element_type=jnp.float32)
        + b[11:12, 0:3])
    return jnp.concatenate([density, color], axis=-1)


def _moe_body(sp_ref, exds_ref, w1_ref, w2_ref, wb_ref, out_ref):
    t = pl.program_id(0)

    @pl.when(t * T < sp_ref[2 * NT])
    def _():
        a = sp_ref[NT + t]
        emb = exds_ref[pl.ds(a, T), :]
        w = w1_ref[0]
        w2 = w2_ref[0]
        b = wb_ref[0]
        out_ref[:] = _mlp_chain(emb, w, w2, b)


def _pad_rows(w, rows):
    return jnp.pad(w, ((0, 0), (0, rows - w.shape[1]), (0, 0)))


def _pad_lanes(bvec, lanes=HX):
    return jnp.pad(bvec, ((0, 0), (0, lanes - bvec.shape[1])))


@jax.jit
def kernel(x, d, index, wx0, bx0, wx1, bx1, wx2, bx2, wx3, bx3, wx4, bx4,
           wx5, bx5, wx6, bx6, wx7, bx7, wint, bint, wden, bden, wc1, bc1,
           wc2, bc2):
    w1 = jnp.concatenate(
        [_pad_rows(wx0, 40), wx1, wx2, wx3, wx4, _pad_rows(wx5, 168),
         wx6, wx7, wint], axis=1)
    w2 = jnp.concatenate(
        [wc1, jnp.pad(wc2, ((0, 0), (0, 0), (0, HD - 3))),
         jnp.pad(wden, ((0, 0), (0, 0), (0, HD - 1)))], axis=1)
    wb = jnp.stack(
        [bx0, bx1, bx2, bx3, bx4, bx5, bx6, bx7, bint,
         _pad_lanes(bden), _pad_lanes(bc1), _pad_lanes(bc2)], axis=1)

    idx = index.astype(jnp.int32)
    order = jnp.argsort(idx).astype(jnp.int32)
    counts = jnp.bincount(idx, length=E).astype(jnp.int32)
    starts = jnp.concatenate(
        [jnp.zeros((1,), jnp.int32), jnp.cumsum(counts)[:-1].astype(jnp.int32)])
    pad_counts = ((counts + T - 1) // T) * T
    pcsum = jnp.cumsum(pad_counts).astype(jnp.int32)
    pad_starts = pcsum - pad_counts

    tile_e = jnp.searchsorted(
        pcsum, jnp.arange(NT, dtype=jnp.int32) * T, side='right').astype(jnp.int32)
    tile_e = jnp.minimum(tile_e, E - 1)
    srow = jnp.clip(
        starts[tile_e] + jnp.arange(NT, dtype=jnp.int32) * T - pad_starts[tile_e],
        0, B)
    sp = jnp.concatenate([tile_e, srow, pcsum[E - 1:E]])

    xd = jnp.concatenate([x, d], axis=1)
    xds = jnp.concatenate([xd[order], jnp.zeros((T, 6), jnp.float32)], axis=0)
    fx = 2.0 ** jnp.arange(NHX, dtype=jnp.float32)
    fd = 2.0 ** jnp.arange(NHD, dtype=jnp.float32)
    px = (xds[:, 0:3, None] * fx).reshape(B + T, 3 * NHX)
    pd = (xds[:, 3:6, None] * fd).reshape(B + T, 3 * NHD)
    exds = jnp.concatenate(
        [jnp.sin(px), jnp.cos(px), jnp.sin(pd), jnp.cos(pd)], axis=1)

    grid_spec = pltpu.PrefetchScalarGridSpec(
        num_scalar_prefetch=1,
        grid=(NT,),
        in_specs=[
            pl.BlockSpec((B + T, 60), lambda t, sp: (0, 0)),
            pl.BlockSpec((1, _R1, HX), lambda t, sp: (sp[t], 0, 0)),
            pl.BlockSpec((1, _R2, HD), lambda t, sp: (sp[t], 0, 0)),
            pl.BlockSpec((1, 12, HX), lambda t, sp: (sp[t], 0, 0)),
        ],
        out_specs=pl.BlockSpec((T, 4), lambda t, sp: (t, 0)),
    )
    outp = pl.pallas_call(
        _moe_body,
        grid_spec=grid_spec,
        out_shape=jax.ShapeDtypeStruct((NS, 4), jnp.float32),
    )(sp, exds, w1, w2, wb)

    sorted_e = idx[order]
    slot = jnp.arange(B, dtype=jnp.int32) + (pad_starts - starts)[sorted_e]
    slot_of_row = jnp.zeros((B,), jnp.int32).at[order].set(slot)
    return outp[slot_of_row]


# speedup vs baseline: 1.2160x; 1.0169x over previous
import functools

import jax
import jax.numpy as jnp
import numpy as np
from jax.experimental import pallas as pl
from jax.experimental.pallas import tpu as pltpu

E = 100
HX = 128
HD = 64
NHX = 6
NHD = 4
B = 4096
DIMX = 3 * NHX * 2
DIMD = 3 * NHD * 2

T = 256
NT = B // T + E
NS = NT * T

_OFF = [0, 40, 168, 296, 424, 552, 720, 848]
_DIN = [DIMX, HX, HX, HX, HX, HX + DIMX, HX, HX]
_OFF_INT = 976
_R1 = 1104
_R2 = 152 + 64 + 128


def _mlp_chain(emb, w, w2, b, nway=2):
    M = emb.shape[0]
    S = M // nway
    exs = [emb[k * S:(k + 1) * S, 0:DIMX] for k in range(nway)]
    eds = [emb[k * S:(k + 1) * S, DIMX:DIMX + DIMD] for k in range(nway)]
    ys = list(exs)
    for li in range(8):
        if li == 5:
            ys = [jnp.concatenate([y, e], axis=-1) for y, e in zip(ys, exs)]
        wli = w[_OFF[li]:_OFF[li] + _DIN[li]]
        hs = [jnp.dot(y, wli, preferred_element_type=jnp.float32) for y in ys]
        ys = [jnp.maximum(h + b[li:li + 1], 0.0) for h in hs]
    inters = [jnp.dot(y, w[_OFF_INT:_OFF_INT + HX],
                      preferred_element_type=jnp.float32) + b[8:9] for y in ys]
    denss = [jnp.dot(y, w2[216:344, 0:1],
                     preferred_element_type=jnp.float32) + b[9:10, 0:1] for y in ys]
    cis = [jnp.concatenate([i, e], axis=-1) for i, e in zip(inters, eds)]
    cs = [jnp.maximum(
        jnp.dot(ci, w2[0:152], preferred_element_type=jnp.float32) + b[10:11, 0:HD],
        0.0) for ci in cis]
    cols = [jax.nn.sigmoid(
        jnp.dot(c, w2[152:216, 0:3], preferred_element_type=jnp.float32)
        + b[11:12, 0:3]) for c in cs]
    return jnp.concatenate(
        [jnp.concatenate([dn, cl], axis=-1) for dn, cl in zip(denss, cols)], axis=0)


def _moe_body(sp_ref, exds_ref, w1_ref, w2_ref, wb_ref, out_ref):
    t = pl.program_id(0)

    @pl.when(t * T < sp_ref[2 * NT])
    def _():
        a = sp_ref[NT + t]
        emb = exds_ref[pl.ds(a, T), :]
        w = w1_ref[0]
        w2 = w2_ref[0]
        b = wb_ref[0]
        out_ref[:] = _mlp_chain(emb, w, w2, b)


def _pad_rows(w, rows):
    return jnp.pad(w, ((0, 0), (0, rows - w.shape[1]), (0, 0)))


def _pad_lanes(bvec, lanes=HX):
    return jnp.pad(bvec, ((0, 0), (0, lanes - bvec.shape[1])))


@jax.jit
def kernel(x, d, index, wx0, bx0, wx1, bx1, wx2, bx2, wx3, bx3, wx4, bx4,
           wx5, bx5, wx6, bx6, wx7, bx7, wint, bint, wden, bden, wc1, bc1,
           wc2, bc2):
    w1 = jnp.concatenate(
        [_pad_rows(wx0, 40), wx1, wx2, wx3, wx4, _pad_rows(wx5, 168),
         wx6, wx7, wint], axis=1)
    w2 = jnp.concatenate(
        [wc1, jnp.pad(wc2, ((0, 0), (0, 0), (0, HD - 3))),
         jnp.pad(wden, ((0, 0), (0, 0), (0, HD - 1)))], axis=1)
    wb = jnp.stack(
        [bx0, bx1, bx2, bx3, bx4, bx5, bx6, bx7, bint,
         _pad_lanes(bden), _pad_lanes(bc1), _pad_lanes(bc2)], axis=1)

    idx = index.astype(jnp.int32)
    order = jnp.argsort(idx).astype(jnp.int32)
    counts = jnp.bincount(idx, length=E).astype(jnp.int32)
    starts = jnp.concatenate(
        [jnp.zeros((1,), jnp.int32), jnp.cumsum(counts)[:-1].astype(jnp.int32)])
    pad_counts = ((counts + T - 1) // T) * T
    pcsum = jnp.cumsum(pad_counts).astype(jnp.int32)
    pad_starts = pcsum - pad_counts

    tile_e = jnp.searchsorted(
        pcsum, jnp.arange(NT, dtype=jnp.int32) * T, side='right').astype(jnp.int32)
    tile_e = jnp.minimum(tile_e, E - 1)
    srow = jnp.clip(
        starts[tile_e] + jnp.arange(NT, dtype=jnp.int32) * T - pad_starts[tile_e],
        0, B)
    sp = jnp.concatenate([tile_e, srow, pcsum[E - 1:E]])

    xd = jnp.concatenate([x, d], axis=1)
    xds = jnp.concatenate([xd[order], jnp.zeros((T, 6), jnp.float32)], axis=0)
    fx = 2.0 ** jnp.arange(NHX, dtype=jnp.float32)
    fd = 2.0 ** jnp.arange(NHD, dtype=jnp.float32)
    px = (xds[:, 0:3, None] * fx).reshape(B + T, 3 * NHX)
    pd = (xds[:, 3:6, None] * fd).reshape(B + T, 3 * NHD)
    exds = jnp.concatenate(
        [jnp.sin(px), jnp.cos(px), jnp.sin(pd), jnp.cos(pd)], axis=1)

    grid_spec = pltpu.PrefetchScalarGridSpec(
        num_scalar_prefetch=1,
        grid=(NT,),
        in_specs=[
            pl.BlockSpec((B + T, 60), lambda t, sp: (0, 0)),
            pl.BlockSpec((1, _R1, HX), lambda t, sp: (sp[t], 0, 0)),
            pl.BlockSpec((1, _R2, HD), lambda t, sp: (sp[t], 0, 0)),
            pl.BlockSpec((1, 12, HX), lambda t, sp: (sp[t], 0, 0)),
        ],
        out_specs=pl.BlockSpec((T, 4), lambda t, sp: (t, 0)),
    )
    outp = pl.pallas_call(
        _moe_body,
        grid_spec=grid_spec,
        out_shape=jax.ShapeDtypeStruct((NS, 4), jnp.float32),
    )(sp, exds, w1, w2, wb)

    sorted_e = idx[order]
    slot = jnp.arange(B, dtype=jnp.int32) + (pad_starts - starts)[sorted_e]
    slot_of_row = jnp.zeros((B,), jnp.int32).at[order].set(slot)
    return outp[slot_of_row]


# speedup vs baseline: 1.4869x; 1.2227x over previous
import functools

import jax
import jax.numpy as jnp
import numpy as np
from jax.experimental import pallas as pl
from jax.experimental.pallas import tpu as pltpu

E = 100
HX = 128
HD = 64
NHX = 6
NHD = 4
B = 4096
DIMX = 3 * NHX * 2
DIMD = 3 * NHD * 2

TE = 128
NTILES = B // TE + E
NG = NTILES // 2
NS = NTILES * TE

_OFF = [0, 40, 168, 296, 424, 552, 720, 848]
_DIN = [DIMX, HX, HX, HX, HX, HX + DIMX, HX, HX]
_OFF_INT = 976
_R1 = 1104
_R2 = 152 + 64 + 128


def _mlp_chains(chains):
    exs = [emb[:, 0:DIMX] for emb, _, _, _ in chains]
    eds = [emb[:, DIMX:DIMX + DIMD] for emb, _, _, _ in chains]
    ws = [w for _, w, _, _ in chains]
    w2s = [w2 for _, _, w2, _ in chains]
    bs = [b for _, _, _, b in chains]
    ys = list(exs)
    for li in range(8):
        if li == 5:
            ys = [jnp.concatenate([y, e], axis=-1) for y, e in zip(ys, exs)]
        hs = [jnp.dot(y, w[_OFF[li]:_OFF[li] + _DIN[li]],
                      preferred_element_type=jnp.float32) for y, w in zip(ys, ws)]
        ys = [jnp.maximum(h + b[li:li + 1], 0.0) for h, b in zip(hs, bs)]
    inters = [jnp.dot(y, w[_OFF_INT:_OFF_INT + HX],
                      preferred_element_type=jnp.float32) + b[8:9]
              for y, w, b in zip(ys, ws, bs)]
    denss = [jnp.dot(y, w2[216:344, 0:1],
                     preferred_element_type=jnp.float32) + b[9:10, 0:1]
             for y, w2, b in zip(ys, w2s, bs)]
    cis = [jnp.concatenate([i, e], axis=-1) for i, e in zip(inters, eds)]
    cs = [jnp.maximum(
        jnp.dot(ci, w2[0:152], preferred_element_type=jnp.float32) + b[10:11, 0:HD],
        0.0) for ci, w2, b in zip(cis, w2s, bs)]
    cols = [jax.nn.sigmoid(
        jnp.dot(c, w2[152:216, 0:3], preferred_element_type=jnp.float32)
        + b[11:12, 0:3]) for c, w2, b in zip(cs, w2s, bs)]
    return [jnp.concatenate([dn, cl], axis=-1) for dn, cl in zip(denss, cols)]


def _moe_body(sp_ref, exds_ref, w1a_ref, w2a_ref, wba_ref,
              w1b_ref, w2b_ref, wbb_ref, out_ref):
    g = pl.program_id(0)

    @pl.when(2 * g * TE < sp_ref[2 * NTILES])
    def _():
        aa = sp_ref[NTILES + 2 * g]
        ab = sp_ref[NTILES + 2 * g + 1]
        emba = exds_ref[pl.ds(aa, TE), :]
        embb = exds_ref[pl.ds(ab, TE), :]
        outs = _mlp_chains([
            (emba, w1a_ref[0], w2a_ref[0], wba_ref[0]),
            (embb, w1b_ref[0], w2b_ref[0], wbb_ref[0]),
        ])
        out_ref[0:TE] = outs[0]
        out_ref[TE:2 * TE] = outs[1]


def _pad_rows(w, rows):
    return jnp.pad(w, ((0, 0), (0, rows - w.shape[1]), (0, 0)))


def _pad_lanes(bvec, lanes=HX):
    return jnp.pad(bvec, ((0, 0), (0, lanes - bvec.shape[1])))


@jax.jit
def kernel(x, d, index, wx0, bx0, wx1, bx1, wx2, bx2, wx3, bx3, wx4, bx4,
           wx5, bx5, wx6, bx6, wx7, bx7, wint, bint, wden, bden, wc1, bc1,
           wc2, bc2):
    w1 = jnp.concatenate(
        [_pad_rows(wx0, 40), wx1, wx2, wx3, wx4, _pad_rows(wx5, 168),
         wx6, wx7, wint], axis=1)
    w2 = jnp.concatenate(
        [wc1, jnp.pad(wc2, ((0, 0), (0, 0), (0, HD - 3))),
         jnp.pad(wden, ((0, 0), (0, 0), (0, HD - 1)))], axis=1)
    wb = jnp.stack(
        [bx0, bx1, bx2, bx3, bx4, bx5, bx6, bx7, bint,
         _pad_lanes(bden), _pad_lanes(bc1), _pad_lanes(bc2)], axis=1)

    idx = index.astype(jnp.int32)
    order = jnp.argsort(idx).astype(jnp.int32)
    counts = jnp.bincount(idx, length=E).astype(jnp.int32)
    starts = jnp.concatenate(
        [jnp.zeros((1,), jnp.int32), jnp.cumsum(counts)[:-1].astype(jnp.int32)])
    pad_counts = ((counts + TE - 1) // TE) * TE
    pcsum = jnp.cumsum(pad_counts).astype(jnp.int32)
    pad_starts = pcsum - pad_counts

    tile_e = jnp.searchsorted(
        pcsum, jnp.arange(NTILES, dtype=jnp.int32) * TE,
        side='right').astype(jnp.int32)
    tile_e = jnp.minimum(tile_e, E - 1)
    srow = jnp.clip(
        starts[tile_e] + jnp.arange(NTILES, dtype=jnp.int32) * TE
        - pad_starts[tile_e], 0, B)
    sp = jnp.concatenate([tile_e, srow, pcsum[E - 1:E]])

    xd = jnp.concatenate([x, d], axis=1)
    xds = jnp.concatenate([xd[order], jnp.zeros((TE, 6), jnp.float32)], axis=0)
    fx = 2.0 ** jnp.arange(NHX, dtype=jnp.float32)
    fd = 2.0 ** jnp.arange(NHD, dtype=jnp.float32)
    px = (xds[:, 0:3, None] * fx).reshape(B + TE, 3 * NHX)
    pd = (xds[:, 3:6, None] * fd).reshape(B + TE, 3 * NHD)
    exds = jnp.concatenate(
        [jnp.sin(px), jnp.cos(px), jnp.sin(pd), jnp.cos(pd)], axis=1)

    grid_spec = pltpu.PrefetchScalarGridSpec(
        num_scalar_prefetch=1,
        grid=(NG,),
        in_specs=[
            pl.BlockSpec((B + TE, 60), lambda g, sp: (0, 0)),
            pl.BlockSpec((1, _R1, HX), lambda g, sp: (sp[2 * g], 0, 0)),
            pl.BlockSpec((1, _R2, HD), lambda g, sp: (sp[2 * g], 0, 0)),
            pl.BlockSpec((1, 12, HX), lambda g, sp: (sp[2 * g], 0, 0)),
            pl.BlockSpec((1, _R1, HX), lambda g, sp: (sp[2 * g + 1], 0, 0)),
            pl.BlockSpec((1, _R2, HD), lambda g, sp: (sp[2 * g + 1], 0, 0)),
            pl.BlockSpec((1, 12, HX), lambda g, sp: (sp[2 * g + 1], 0, 0)),
        ],
        out_specs=pl.BlockSpec((2 * TE, 4), lambda g, sp: (g, 0)),
    )
    outp = pl.pallas_call(
        _moe_body,
        grid_spec=grid_spec,
        out_shape=jax.ShapeDtypeStruct((NS, 4), jnp.float32),
    )(sp, exds, w1, w2, wb, w1, w2, wb)

    sorted_e = idx[order]
    slot = jnp.arange(B, dtype=jnp.int32) + (pad_starts - starts)[sorted_e]
    slot_of_row = jnp.zeros((B,), jnp.int32).at[order].set(slot)
    return outp[slot_of_row]


# speedup vs baseline: 1.6896x; 1.1363x over previous
import functools

import jax
import jax.numpy as jnp
import numpy as np
from jax.experimental import pallas as pl
from jax.experimental.pallas import tpu as pltpu

E = 100
HX = 128
HD = 64
NHX = 6
NHD = 4
B = 4096
DIMX = 3 * NHX * 2
DIMD = 3 * NHD * 2

TE = 128
NTILES = B // TE + E
NEX = 4
NG = NTILES // NEX
NS = NTILES * TE

_OFF = [0, 40, 168, 296, 424, 552, 720, 848]
_DIN = [DIMX, HX, HX, HX, HX, HX + DIMX, HX, HX]
_OFF_INT = 976
_R1 = 1104
_R2 = 152 + 64 + 128


def _mlp_chains(chains):
    exs = [emb[:, 0:DIMX] for emb, _, _, _ in chains]
    eds = [emb[:, DIMX:DIMX + DIMD] for emb, _, _, _ in chains]
    ws = [w for _, w, _, _ in chains]
    w2s = [w2 for _, _, w2, _ in chains]
    bs = [b for _, _, _, b in chains]
    ys = list(exs)
    for li in range(8):
        if li == 5:
            ys = [jnp.concatenate([y, e], axis=-1) for y, e in zip(ys, exs)]
        hs = [jnp.dot(y, w[_OFF[li]:_OFF[li] + _DIN[li]],
                      preferred_element_type=jnp.float32) for y, w in zip(ys, ws)]
        ys = [jnp.maximum(h + b[li:li + 1], 0.0) for h, b in zip(hs, bs)]
    inters = [jnp.dot(y, w[_OFF_INT:_OFF_INT + HX],
                      preferred_element_type=jnp.float32) + b[8:9]
              for y, w, b in zip(ys, ws, bs)]
    denss = [jnp.dot(y, w2[216:344, 0:1],
                     preferred_element_type=jnp.float32) + b[9:10, 0:1]
             for y, w2, b in zip(ys, w2s, bs)]
    cis = [jnp.concatenate([i, e], axis=-1) for i, e in zip(inters, eds)]
    cs = [jnp.maximum(
        jnp.dot(ci, w2[0:152], preferred_element_type=jnp.float32) + b[10:11, 0:HD],
        0.0) for ci, w2, b in zip(cis, w2s, bs)]
    cols = [jax.nn.sigmoid(
        jnp.dot(c, w2[152:216, 0:3], preferred_element_type=jnp.float32)
        + b[11:12, 0:3]) for c, w2, b in zip(cs, w2s, bs)]
    return [jnp.concatenate([dn, cl], axis=-1) for dn, cl in zip(denss, cols)]


def _moe_body(sp_ref, exds_ref, *refs):
    out_ref = refs[-1]
    wrefs = refs[:-1]
    g = pl.program_id(0)

    @pl.when(NEX * g * TE < sp_ref[2 * NTILES])
    def _():
        chains = []
        for j in range(NEX):
            a = sp_ref[NTILES + NEX * g + j]
            emb = exds_ref[pl.ds(a, TE), :]
            chains.append((emb, wrefs[3 * j][0], wrefs[3 * j + 1][0],
                           wrefs[3 * j + 2][0]))
        outs = _mlp_chains(chains)
        for j in range(NEX):
            out_ref[j * TE:(j + 1) * TE] = outs[j]


def _pad_rows(w, rows):
    return jnp.pad(w, ((0, 0), (0, rows - w.shape[1]), (0, 0)))


def _pad_lanes(bvec, lanes=HX):
    return jnp.pad(bvec, ((0, 0), (0, lanes - bvec.shape[1])))


@jax.jit
def kernel(x, d, index, wx0, bx0, wx1, bx1, wx2, bx2, wx3, bx3, wx4, bx4,
           wx5, bx5, wx6, bx6, wx7, bx7, wint, bint, wden, bden, wc1, bc1,
           wc2, bc2):
    w1 = jnp.concatenate(
        [_pad_rows(wx0, 40), wx1, wx2, wx3, wx4, _pad_rows(wx5, 168),
         wx6, wx7, wint], axis=1)
    w2 = jnp.concatenate(
        [wc1, jnp.pad(wc2, ((0, 0), (0, 0), (0, HD - 3))),
         jnp.pad(wden, ((0, 0), (0, 0), (0, HD - 1)))], axis=1)
    wb = jnp.stack(
        [bx0, bx1, bx2, bx3, bx4, bx5, bx6, bx7, bint,
         _pad_lanes(bden), _pad_lanes(bc1), _pad_lanes(bc2)], axis=1)

    idx = index.astype(jnp.int32)
    order = jnp.argsort(idx).astype(jnp.int32)
    counts = jnp.bincount(idx, length=E).astype(jnp.int32)
    starts = jnp.concatenate(
        [jnp.zeros((1,), jnp.int32), jnp.cumsum(counts)[:-1].astype(jnp.int32)])
    pad_counts = ((counts + TE - 1) // TE) * TE
    pcsum = jnp.cumsum(pad_counts).astype(jnp.int32)
    pad_starts = pcsum - pad_counts

    tile_e = jnp.searchsorted(
        pcsum, jnp.arange(NTILES, dtype=jnp.int32) * TE,
        side='right').astype(jnp.int32)
    tile_e = jnp.minimum(tile_e, E - 1)
    srow = jnp.clip(
        starts[tile_e] + jnp.arange(NTILES, dtype=jnp.int32) * TE
        - pad_starts[tile_e], 0, B)
    sp = jnp.concatenate([tile_e, srow, pcsum[E - 1:E]])

    xd = jnp.concatenate([x, d], axis=1)
    xds = jnp.concatenate([xd[order], jnp.zeros((TE, 6), jnp.float32)], axis=0)
    fx = 2.0 ** jnp.arange(NHX, dtype=jnp.float32)
    fd = 2.0 ** jnp.arange(NHD, dtype=jnp.float32)
    px = (xds[:, 0:3, None] * fx).reshape(B + TE, 3 * NHX)
    pd = (xds[:, 3:6, None] * fd).reshape(B + TE, 3 * NHD)
    exds = jnp.concatenate(
        [jnp.sin(px), jnp.cos(px), jnp.sin(pd), jnp.cos(pd)], axis=1)

    grid_spec = pltpu.PrefetchScalarGridSpec(
        num_scalar_prefetch=1,
        grid=(NG,),
        in_specs=[pl.BlockSpec((B + TE, 60), lambda g, sp: (0, 0))] + sum(
            [[pl.BlockSpec((1, _R1, HX),
                           functools.partial(
                               lambda j, g, sp: (sp[NEX * g + j], 0, 0), j)),
              pl.BlockSpec((1, _R2, HD),
                           functools.partial(
                               lambda j, g, sp: (sp[NEX * g + j], 0, 0), j)),
              pl.BlockSpec((1, 12, HX),
                           functools.partial(
                               lambda j, g, sp: (sp[NEX * g + j], 0, 0), j))]
             for j in range(NEX)], []),
        out_specs=pl.BlockSpec((NEX * TE, 4), lambda g, sp: (g, 0)),
    )
    outp = pl.pallas_call(
        _moe_body,
        grid_spec=grid_spec,
        out_shape=jax.ShapeDtypeStruct((NS, 4), jnp.float32),
    )(sp, exds, *([w1, w2, wb] * NEX))

    sorted_e = idx[order]
    slot = jnp.arange(B, dtype=jnp.int32) + (pad_starts - starts)[sorted_e]
    slot_of_row = jnp.zeros((B,), jnp.int32).at[order].set(slot)
    return outp[slot_of_row]
